# Initial kernel scaffold; baseline (speedup 1.0000x reference)
import jax
import jax.numpy as jnp
from jax.experimental import pallas as pl


def kernel(x, edge_index, edge_weight, W1, b1, W2, b2, W3, b3, W4, b4):
    raise NotImplementedError("write your pallas kernel here")


# speedup vs baseline: 10.2999x; 10.2999x over previous
import functools
import jax
import jax.numpy as jnp
from jax import lax
from jax.experimental import pallas as pl
from jax.experimental.pallas import tpu as pltpu
from jax.experimental.pallas import tpu_sc as plsc

N = 10000
NPAD = 10240
E = 320000
D = 16
NC = 2
NS = 16
TILES = NC * NS
EPT = E // TILES
CH = 80
NCH = EPT // CH
RPS = NPAD // NS

_mesh = plsc.VectorSubcoreMesh(core_axis_name="c", subcore_axis_name="s")


@functools.partial(
    pl.kernel, mesh=_mesh,
    compiler_params=pltpu.CompilerParams(use_tc_tiling_on_sc=False),
    out_type=jax.ShapeDtypeStruct((NC, NPAD), jnp.float32),
    scratch_types=[
        pltpu.VMEM((CH,), jnp.int32),
        pltpu.VMEM((CH,), jnp.float32),
        pltpu.VMEM_SHARED((NPAD,), jnp.float32),
    ],
)
def _sc_degree(col_hbm, w_hbm, z_hbm, out_hbm, idx_v, w_v, acc):
    c = lax.axis_index("c")
    s = lax.axis_index("s")
    wid = s * NC + c
    pltpu.sync_copy(z_hbm.at[pl.ds(s * RPS, RPS)], acc.at[pl.ds(s * RPS, RPS)])
    plsc.subcore_barrier()
    base = wid * EPT

    def body(ch, _):
        off = base + ch * CH
        pltpu.sync_copy(col_hbm.at[pl.ds(off, CH)], idx_v)
        pltpu.sync_copy(w_hbm.at[pl.ds(off, CH)], w_v)
        pltpu.sync_copy(w_v, acc.at[idx_v], add=True)
        return _

    lax.fori_loop(0, NCH, body, 0)
    plsc.subcore_barrier()
    pltpu.sync_copy(acc.at[pl.ds(s * RPS, RPS)],
                    out_hbm.at[c, pl.ds(s * RPS, RPS)])


@functools.partial(
    pl.kernel, mesh=_mesh,
    compiler_params=pltpu.CompilerParams(use_tc_tiling_on_sc=False),
    out_type=jax.ShapeDtypeStruct((NC, NPAD, D), jnp.float32),
    scratch_types=[
        pltpu.VMEM((CH,), jnp.int32),
        pltpu.VMEM((CH,), jnp.int32),
        pltpu.VMEM((CH,), jnp.float32),
        pltpu.VMEM((CH, D), jnp.float32),
        pltpu.VMEM_SHARED((NPAD, D), jnp.float32),
        pltpu.SemaphoreType.DMA,
    ],
)
def _sc_edge(hs_hbm, row_hbm, col_hbm, w_hbm, z_hbm, out_hbm,
             idxg_v, idxs_v, w_v, rows_v, acc, sem):
    c = lax.axis_index("c")
    s = lax.axis_index("s")
    wid = s * NC + c
    pltpu.sync_copy(z_hbm.at[pl.ds(s * RPS, RPS)], acc.at[pl.ds(s * RPS, RPS)])
    plsc.subcore_barrier()
    base = wid * EPT

    def body(ch, _):
        off = base + ch * CH
        pltpu.sync_copy(row_hbm.at[pl.ds(off, CH)], idxg_v)
        pltpu.sync_copy(col_hbm.at[pl.ds(off, CH)], idxs_v)
        pltpu.sync_copy(w_hbm.at[pl.ds(off, CH)], w_v)
        pltpu.async_copy(hs_hbm.at[idxg_v], rows_v, sem).wait()

        def scale(j, carry):
            wvec = w_v[pl.ds(j * 16, 16)]
            for k in range(16):
                i = j * 16 + k
                rows_v[i, :] = rows_v[i, :] * wvec[k]
            return carry

        lax.fori_loop(0, CH // 16, scale, 0)
        pltpu.sync_copy(rows_v, acc.at[idxs_v], add=True)
        return _

    lax.fori_loop(0, NCH, body, 0)
    plsc.subcore_barrier()
    pltpu.sync_copy(acc.at[pl.ds(s * RPS, RPS)],
                    out_hbm.at[c, pl.ds(s * RPS, RPS)])


def _pre_body(x_ref, w1_ref, p0_ref, p1_ref, m_ref, dinv_ref, hs_ref):
    deg = p0_ref[...] + p1_ref[...] + m_ref[...]
    dinv = jnp.where(deg > 0, lax.rsqrt(deg), 0.0)
    dinv_ref[...] = dinv
    h = jnp.dot(x_ref[...], w1_ref[...], preferred_element_type=jnp.float32)
    hs_ref[...] = dinv * h


def _tc_pre(x_pad, w1, p0, p1, m):
    return pl.pallas_call(
        _pre_body,
        out_shape=[jax.ShapeDtypeStruct((NPAD, 1), jnp.float32),
                   jax.ShapeDtypeStruct((NPAD, D), jnp.float32)],
    )(x_pad, w1, p0, p1, m)


def _mid_body(a0_ref, a1_ref, hs_ref, dinv_ref, b_ref, wn_ref, out_ref):
    dinv = dinv_ref[...]
    y = jnp.maximum(
        dinv * (a0_ref[...] + a1_ref[...] + hs_ref[...]) + b_ref[...], 0.0)
    out_ref[...] = dinv * jnp.dot(y, wn_ref[...],
                                  preferred_element_type=jnp.float32)


def _tc_mid(a0, a1, hs, dinv, b, wn):
    return pl.pallas_call(
        _mid_body,
        out_shape=jax.ShapeDtypeStruct((NPAD, D), jnp.float32),
    )(a0, a1, hs, dinv, b, wn)


def _fin_body(a0_ref, a1_ref, hs_ref, dinv_ref, b_ref, out_ref):
    out_ref[...] = jnp.maximum(
        dinv_ref[...] * (a0_ref[...] + a1_ref[...] + hs_ref[...])
        + b_ref[...], 0.0)


def _tc_fin(a0, a1, hs, dinv, b):
    return pl.pallas_call(
        _fin_body,
        out_shape=jax.ShapeDtypeStruct((NPAD, D), jnp.float32),
    )(a0, a1, hs, dinv, b)


def _pad16(w):
    out = jnp.zeros((D, D), jnp.float32)
    return out.at[:w.shape[0], :w.shape[1]].set(w)


def kernel(x, edge_index, edge_weight, W1, b1, W2, b2, W3, b3, W4, b4):
    row = edge_index[0]
    col = edge_index[1]
    x_pad = jnp.zeros((NPAD, 128), jnp.float32).at[:N].set(x)
    z_n = jnp.zeros((NPAD,), jnp.float32)
    z_nd = jnp.zeros((NPAD, D), jnp.float32)
    m = (jnp.arange(NPAD) < N).astype(jnp.float32)[:, None]

    degp = _sc_degree(col, edge_weight, z_n)
    p0 = degp[0][:, None]
    p1 = degp[1][:, None]
    dinv, hs = _tc_pre(x_pad, W1, p0, p1, m)

    w2p, w3p, w4p = _pad16(W2), _pad16(W3), _pad16(W4)
    b1p = b1[None, :]
    b2p = jnp.zeros((1, D), jnp.float32).at[0, :b2.shape[0]].set(b2)
    b3p = jnp.zeros((1, D), jnp.float32).at[0, :b3.shape[0]].set(b3)
    b4p = jnp.zeros((1, D), jnp.float32).at[0, :b4.shape[0]].set(b4)

    for bi, wn in ((b1p, w2p), (b2p, w3p), (b3p, w4p)):
        acc = _sc_edge(hs, row, col, edge_weight, z_nd)
        hs = _tc_mid(acc[0], acc[1], hs, dinv, bi, wn)

    acc = _sc_edge(hs, row, col, edge_weight, z_nd)
    y = _tc_fin(acc[0], acc[1], hs, dinv, b4p)
    return y[:N, :8]


# speedup vs baseline: 31.7845x; 3.0859x over previous
import functools
import jax
import jax.numpy as jnp
from jax import lax
from jax.experimental import pallas as pl
from jax.experimental.pallas import tpu as pltpu
from jax.experimental.pallas import tpu_sc as plsc

N = 10000
NPAD = 10240
E = 320000
D = 16
NC = 2
NS = 16
TILES = NC * NS
EPT = E // TILES
CH = 80
NCH = EPT // CH
RPS = NPAD // NS

_mesh = plsc.VectorSubcoreMesh(core_axis_name="c", subcore_axis_name="s")


@functools.partial(
    pl.kernel, mesh=_mesh,
    compiler_params=pltpu.CompilerParams(use_tc_tiling_on_sc=False),
    out_type=jax.ShapeDtypeStruct((NC, NPAD), jnp.float32),
    scratch_types=[
        pltpu.VMEM((NCH, CH), jnp.int32),
        pltpu.VMEM((NCH, CH), jnp.float32),
        pltpu.VMEM_SHARED((NPAD,), jnp.float32),
    ],
)
def _sc_degree(col_hbm, w_hbm, z_hbm, out_hbm, idx_v, w_v, acc):
    c = lax.axis_index("c")
    s = lax.axis_index("s")
    wid = s * NC + c
    pltpu.sync_copy(z_hbm.at[pl.ds(s * RPS, RPS)], acc.at[pl.ds(s * RPS, RPS)])
    pltpu.sync_copy(col_hbm.at[wid], idx_v)
    pltpu.sync_copy(w_hbm.at[wid], w_v)
    plsc.subcore_barrier()

    def body(ch, _):
        pltpu.sync_copy(w_v.at[ch], acc.at[idx_v.at[ch]], add=True)
        return _

    lax.fori_loop(0, NCH, body, 0)
    plsc.subcore_barrier()
    pltpu.sync_copy(acc.at[pl.ds(s * RPS, RPS)],
                    out_hbm.at[c, pl.ds(s * RPS, RPS)])


@functools.partial(
    pl.kernel, mesh=_mesh,
    compiler_params=pltpu.CompilerParams(use_tc_tiling_on_sc=False),
    out_type=jax.ShapeDtypeStruct((NC, NPAD, D), jnp.float32),
    scratch_types=[
        pltpu.VMEM((NCH, CH), jnp.int32),
        pltpu.VMEM((NCH, CH), jnp.int32),
        pltpu.VMEM((NCH, CH), jnp.float32),
        pltpu.VMEM((CH, D), jnp.float32),
        pltpu.VMEM((CH, D), jnp.float32),
        pltpu.VMEM_SHARED((NPAD, D), jnp.float32),
        pltpu.SemaphoreType.DMA,
        pltpu.SemaphoreType.DMA,
    ],
)
def _sc_edge(hs_hbm, row_hbm, col_hbm, w_hbm, z_hbm, out_hbm,
             idxg_v, idxs_v, w_v, rows_a, rows_b, acc, sem_a, sem_b):
    c = lax.axis_index("c")
    s = lax.axis_index("s")
    wid = s * NC + c
    pltpu.sync_copy(z_hbm.at[pl.ds(s * RPS, RPS)], acc.at[pl.ds(s * RPS, RPS)])
    pltpu.sync_copy(row_hbm.at[wid], idxg_v)
    pltpu.sync_copy(col_hbm.at[wid], idxs_v)
    pltpu.sync_copy(w_hbm.at[wid], w_v)
    plsc.subcore_barrier()

    def gather_start(ch, buf, sem):
        pltpu.async_copy(hs_hbm.at[idxg_v.at[ch]], buf, sem)

    def gather_wait(ch, buf, sem):
        pltpu.make_async_copy(hs_hbm.at[idxg_v.at[ch]], buf, sem).wait()

    def process(ch, buf):
        for g in range(CH // 16):
            wvec = w_v[ch, pl.ds(g * 16, 16)]
            for k in range(16):
                i = g * 16 + k
                buf[i, :] = buf[i, :] * wvec[k]
        pltpu.sync_copy(buf, acc.at[idxs_v.at[ch]], add=True)

    gather_start(0, rows_a, sem_a)

    def pair(t, _):
        ch0 = 2 * t
        gather_start(ch0 + 1, rows_b, sem_b)
        gather_wait(ch0, rows_a, sem_a)
        process(ch0, rows_a)
        gather_start(ch0 + 2, rows_a, sem_a)
        gather_wait(ch0 + 1, rows_b, sem_b)
        process(ch0 + 1, rows_b)
        return _

    lax.fori_loop(0, (NCH - 1) // 2, pair, 0)
    gather_wait(NCH - 1, rows_a, sem_a)
    process(NCH - 1, rows_a)
    plsc.subcore_barrier()
    pltpu.sync_copy(acc.at[pl.ds(s * RPS, RPS)],
                    out_hbm.at[c, pl.ds(s * RPS, RPS)])


def _pre_body(x_ref, w1_ref, p0_ref, p1_ref, m_ref, dinv_ref, hs_ref):
    deg = p0_ref[...] + p1_ref[...] + m_ref[...]
    dinv = jnp.where(deg > 0, lax.rsqrt(deg), 0.0)
    dinv_ref[...] = dinv
    h = jnp.dot(x_ref[...], w1_ref[...], preferred_element_type=jnp.float32)
    hs_ref[...] = dinv * h


def _tc_pre(x_pad, w1, p0, p1, m):
    return pl.pallas_call(
        _pre_body,
        out_shape=[jax.ShapeDtypeStruct((NPAD, 1), jnp.float32),
                   jax.ShapeDtypeStruct((NPAD, D), jnp.float32)],
    )(x_pad, w1, p0, p1, m)


def _mid_body(a0_ref, a1_ref, hs_ref, dinv_ref, b_ref, wn_ref, out_ref):
    dinv = dinv_ref[...]
    y = jnp.maximum(
        dinv * (a0_ref[...] + a1_ref[...] + hs_ref[...]) + b_ref[...], 0.0)
    out_ref[...] = dinv * jnp.dot(y, wn_ref[...],
                                  preferred_element_type=jnp.float32)


def _tc_mid(a0, a1, hs, dinv, b, wn):
    return pl.pallas_call(
        _mid_body,
        out_shape=jax.ShapeDtypeStruct((NPAD, D), jnp.float32),
    )(a0, a1, hs, dinv, b, wn)


def _fin_body(a0_ref, a1_ref, hs_ref, dinv_ref, b_ref, out_ref):
    out_ref[...] = jnp.maximum(
        dinv_ref[...] * (a0_ref[...] + a1_ref[...] + hs_ref[...])
        + b_ref[...], 0.0)


def _tc_fin(a0, a1, hs, dinv, b):
    return pl.pallas_call(
        _fin_body,
        out_shape=jax.ShapeDtypeStruct((NPAD, D), jnp.float32),
    )(a0, a1, hs, dinv, b)


def _pad16(w):
    out = jnp.zeros((D, D), jnp.float32)
    return out.at[:w.shape[0], :w.shape[1]].set(w)


def kernel(x, edge_index, edge_weight, W1, b1, W2, b2, W3, b3, W4, b4):
    row = edge_index[0].reshape(TILES, NCH, CH)
    col = edge_index[1].reshape(TILES, NCH, CH)
    ew = edge_weight.reshape(TILES, NCH, CH)
    x_pad = jnp.zeros((NPAD, 128), jnp.float32).at[:N].set(x)
    z_n = jnp.zeros((NPAD,), jnp.float32)
    z_nd = jnp.zeros((NPAD, D), jnp.float32)
    m = (jnp.arange(NPAD) < N).astype(jnp.float32)[:, None]

    degp = _sc_degree(col, ew, z_n)
    p0 = degp[0][:, None]
    p1 = degp[1][:, None]
    dinv, hs = _tc_pre(x_pad, W1, p0, p1, m)

    w2p, w3p, w4p = _pad16(W2), _pad16(W3), _pad16(W4)
    b1p = b1[None, :]
    b2p = jnp.zeros((1, D), jnp.float32).at[0, :b2.shape[0]].set(b2)
    b3p = jnp.zeros((1, D), jnp.float32).at[0, :b3.shape[0]].set(b3)
    b4p = jnp.zeros((1, D), jnp.float32).at[0, :b4.shape[0]].set(b4)

    for bi, wn in ((b1p, w2p), (b2p, w3p), (b3p, w4p)):
        acc = _sc_edge(hs, row, col, ew, z_nd)
        hs = _tc_mid(acc[0], acc[1], hs, dinv, bi, wn)

    acc = _sc_edge(hs, row, col, ew, z_nd)
    y = _tc_fin(acc[0], acc[1], hs, dinv, b4p)
    return y[:N, :8]


# speedup vs baseline: 43.9340x; 1.3822x over previous
import functools
import jax
import jax.numpy as jnp
from jax import lax
from jax.experimental import pallas as pl
from jax.experimental.pallas import tpu as pltpu
from jax.experimental.pallas import tpu_sc as plsc

N = 10000
NPAD = 10240
E = 320000
D = 16
NC = 2
NS = 16
TILES = NC * NS
EPT = E // TILES
CH = 80
NCH = EPT // CH
RPS = NPAD // NS

_mesh = plsc.VectorSubcoreMesh(core_axis_name="c", subcore_axis_name="s")


@functools.partial(
    pl.kernel, mesh=_mesh,
    compiler_params=pltpu.CompilerParams(use_tc_tiling_on_sc=False),
    out_type=jax.ShapeDtypeStruct((NC, NPAD), jnp.float32),
    scratch_types=[
        pltpu.VMEM((NCH, CH), jnp.int32),
        pltpu.VMEM((NCH, CH), jnp.float32),
        pltpu.VMEM_SHARED((NPAD,), jnp.float32),
    ],
)
def _sc_degree(col_hbm, w_hbm, z_hbm, out_hbm, idx_v, w_v, acc):
    c = lax.axis_index("c")
    s = lax.axis_index("s")
    wid = s * NC + c
    pltpu.sync_copy(z_hbm.at[pl.ds(s * RPS, RPS)], acc.at[pl.ds(s * RPS, RPS)])
    pltpu.sync_copy(col_hbm.at[wid], idx_v)
    pltpu.sync_copy(w_hbm.at[wid], w_v)
    plsc.subcore_barrier()

    def body(ch, _):
        pltpu.sync_copy(w_v.at[ch], acc.at[idx_v.at[ch]], add=True)
        return _

    lax.fori_loop(0, NCH, body, 0)
    plsc.subcore_barrier()
    pltpu.sync_copy(acc.at[pl.ds(s * RPS, RPS)],
                    out_hbm.at[c, pl.ds(s * RPS, RPS)])


@functools.partial(
    pl.kernel, mesh=_mesh,
    compiler_params=pltpu.CompilerParams(use_tc_tiling_on_sc=False),
    out_type=jax.ShapeDtypeStruct((NC, NPAD, D), jnp.float32),
    scratch_types=[
        pltpu.VMEM((NCH, CH), jnp.int32),
        pltpu.VMEM((NCH, CH), jnp.int32),
        pltpu.VMEM((NCH, CH), jnp.float32),
        pltpu.VMEM((CH, D), jnp.float32),
        pltpu.VMEM((CH, D), jnp.float32),
        pltpu.VMEM_SHARED((NPAD, D), jnp.float32),
        pltpu.VMEM_SHARED((NPAD, D), jnp.float32),
        pltpu.SemaphoreType.DMA,
        pltpu.SemaphoreType.DMA,
    ],
)
def _sc_edge(hs_hbm, row_hbm, col_hbm, w_hbm, z_hbm, out_hbm,
             idxg_v, idxs_v, w_v, rows_a, rows_b, acc, hs_sp, sem_a, sem_b):
    c = lax.axis_index("c")
    s = lax.axis_index("s")
    wid = s * NC + c
    pltpu.sync_copy(z_hbm.at[pl.ds(s * RPS, RPS)], acc.at[pl.ds(s * RPS, RPS)])
    pltpu.sync_copy(hs_hbm.at[pl.ds(s * RPS, RPS)],
                    hs_sp.at[pl.ds(s * RPS, RPS)])
    pltpu.sync_copy(row_hbm.at[wid], idxg_v)
    pltpu.sync_copy(col_hbm.at[wid], idxs_v)
    pltpu.sync_copy(w_hbm.at[wid], w_v)
    plsc.subcore_barrier()

    def gather_start(ch, buf, sem):
        pltpu.async_copy(hs_sp.at[idxg_v.at[ch]], buf, sem)

    def gather_wait(ch, buf, sem):
        pltpu.make_async_copy(hs_sp.at[idxg_v.at[ch]], buf, sem).wait()

    def process(ch, buf):
        for g in range(CH // 16):
            wvec = w_v[ch, pl.ds(g * 16, 16)]
            for k in range(16):
                i = g * 16 + k
                buf[i, :] = buf[i, :] * wvec[k]
        pltpu.sync_copy(buf, acc.at[idxs_v.at[ch]], add=True)

    gather_start(0, rows_a, sem_a)

    def pair(t, _):
        ch0 = 2 * t
        gather_start(ch0 + 1, rows_b, sem_b)
        gather_wait(ch0, rows_a, sem_a)
        process(ch0, rows_a)
        gather_start(ch0 + 2, rows_a, sem_a)
        gather_wait(ch0 + 1, rows_b, sem_b)
        process(ch0 + 1, rows_b)
        return _

    lax.fori_loop(0, (NCH - 1) // 2, pair, 0)
    gather_wait(NCH - 1, rows_a, sem_a)
    process(NCH - 1, rows_a)
    plsc.subcore_barrier()
    pltpu.sync_copy(acc.at[pl.ds(s * RPS, RPS)],
                    out_hbm.at[c, pl.ds(s * RPS, RPS)])


def _pre_body(x_ref, w1_ref, p0_ref, p1_ref, m_ref, dinv_ref, hs_ref):
    deg = p0_ref[...] + p1_ref[...] + m_ref[...]
    dinv = jnp.where(deg > 0, lax.rsqrt(deg), 0.0)
    dinv_ref[...] = dinv
    h = jnp.dot(x_ref[...], w1_ref[...], preferred_element_type=jnp.float32)
    hs_ref[...] = dinv * h


def _tc_pre(x_pad, w1, p0, p1, m):
    return pl.pallas_call(
        _pre_body,
        out_shape=[jax.ShapeDtypeStruct((NPAD, 1), jnp.float32),
                   jax.ShapeDtypeStruct((NPAD, D), jnp.float32)],
    )(x_pad, w1, p0, p1, m)


def _mid_body(a0_ref, a1_ref, hs_ref, dinv_ref, b_ref, wn_ref, out_ref):
    dinv = dinv_ref[...]
    y = jnp.maximum(
        dinv * (a0_ref[...] + a1_ref[...] + hs_ref[...]) + b_ref[...], 0.0)
    out_ref[...] = dinv * jnp.dot(y, wn_ref[...],
                                  preferred_element_type=jnp.float32)


def _tc_mid(a0, a1, hs, dinv, b, wn):
    return pl.pallas_call(
        _mid_body,
        out_shape=jax.ShapeDtypeStruct((NPAD, D), jnp.float32),
    )(a0, a1, hs, dinv, b, wn)


def _fin_body(a0_ref, a1_ref, hs_ref, dinv_ref, b_ref, out_ref):
    out_ref[...] = jnp.maximum(
        dinv_ref[...] * (a0_ref[...] + a1_ref[...] + hs_ref[...])
        + b_ref[...], 0.0)


def _tc_fin(a0, a1, hs, dinv, b):
    return pl.pallas_call(
        _fin_body,
        out_shape=jax.ShapeDtypeStruct((NPAD, D), jnp.float32),
    )(a0, a1, hs, dinv, b)


def _pad16(w):
    out = jnp.zeros((D, D), jnp.float32)
    return out.at[:w.shape[0], :w.shape[1]].set(w)


def kernel(x, edge_index, edge_weight, W1, b1, W2, b2, W3, b3, W4, b4):
    row = edge_index[0].reshape(TILES, NCH, CH)
    col = edge_index[1].reshape(TILES, NCH, CH)
    ew = edge_weight.reshape(TILES, NCH, CH)
    x_pad = jnp.zeros((NPAD, 128), jnp.float32).at[:N].set(x)
    z_n = jnp.zeros((NPAD,), jnp.float32)
    z_nd = jnp.zeros((NPAD, D), jnp.float32)
    m = (jnp.arange(NPAD) < N).astype(jnp.float32)[:, None]

    degp = _sc_degree(col, ew, z_n)
    p0 = degp[0][:, None]
    p1 = degp[1][:, None]
    dinv, hs = _tc_pre(x_pad, W1, p0, p1, m)

    w2p, w3p, w4p = _pad16(W2), _pad16(W3), _pad16(W4)
    b1p = b1[None, :]
    b2p = jnp.zeros((1, D), jnp.float32).at[0, :b2.shape[0]].set(b2)
    b3p = jnp.zeros((1, D), jnp.float32).at[0, :b3.shape[0]].set(b3)
    b4p = jnp.zeros((1, D), jnp.float32).at[0, :b4.shape[0]].set(b4)

    for bi, wn in ((b1p, w2p), (b2p, w3p), (b3p, w4p)):
        acc = _sc_edge(hs, row, col, ew, z_nd)
        hs = _tc_mid(acc[0], acc[1], hs, dinv, bi, wn)

    acc = _sc_edge(hs, row, col, ew, z_nd)
    y = _tc_fin(acc[0], acc[1], hs, dinv, b4p)
    return y[:N, :8]


# speedup vs baseline: 44.1505x; 1.0049x over previous
import functools
import jax
import jax.numpy as jnp
from jax import lax
from jax.experimental import pallas as pl
from jax.experimental.pallas import tpu as pltpu
from jax.experimental.pallas import tpu_sc as plsc

N = 10000
NPAD = 10240
E = 320000
D = 16
NC = 2
NS = 16
TILES = NC * NS
EPT = E // TILES
CH = 128
NCH = 79
EPT_PAD = NCH * CH
RPS = NPAD // NS

_mesh = plsc.VectorSubcoreMesh(core_axis_name="c", subcore_axis_name="s")


@functools.partial(
    pl.kernel, mesh=_mesh,
    compiler_params=pltpu.CompilerParams(use_tc_tiling_on_sc=False),
    out_type=jax.ShapeDtypeStruct((NC, NPAD), jnp.float32),
    scratch_types=[
        pltpu.VMEM((NCH, CH), jnp.int32),
        pltpu.VMEM((NCH, CH), jnp.float32),
        pltpu.VMEM_SHARED((NPAD,), jnp.float32),
    ],
)
def _sc_degree(col_hbm, w_hbm, z_hbm, out_hbm, idx_v, w_v, acc):
    c = lax.axis_index("c")
    s = lax.axis_index("s")
    wid = s * NC + c
    pltpu.sync_copy(z_hbm.at[pl.ds(s * RPS, RPS)], acc.at[pl.ds(s * RPS, RPS)])
    pltpu.sync_copy(col_hbm.at[wid], idx_v)
    pltpu.sync_copy(w_hbm.at[wid], w_v)
    plsc.subcore_barrier()

    def body(ch, _):
        pltpu.sync_copy(w_v.at[ch], acc.at[idx_v.at[ch]], add=True)
        return _

    lax.fori_loop(0, NCH, body, 0)
    plsc.subcore_barrier()
    pltpu.sync_copy(acc.at[pl.ds(s * RPS, RPS)],
                    out_hbm.at[c, pl.ds(s * RPS, RPS)])


@functools.partial(
    pl.kernel, mesh=_mesh,
    compiler_params=pltpu.CompilerParams(use_tc_tiling_on_sc=False),
    out_type=jax.ShapeDtypeStruct((NC, NPAD, D), jnp.float32),
    scratch_types=[
        pltpu.VMEM((NCH, CH), jnp.int32),
        pltpu.VMEM((NCH, CH), jnp.int32),
        pltpu.VMEM((NCH, CH), jnp.float32),
        pltpu.VMEM((CH, D), jnp.float32),
        pltpu.VMEM((CH, D), jnp.float32),
        pltpu.VMEM_SHARED((NPAD, D), jnp.float32),
        pltpu.VMEM_SHARED((NPAD, D), jnp.float32),
        pltpu.SemaphoreType.DMA,
        pltpu.SemaphoreType.DMA,
    ],
)
def _sc_edge(hs_hbm, row_hbm, col_hbm, w_hbm, z_hbm, out_hbm,
             idxg_v, idxs_v, w_v, rows_a, rows_b, acc, hs_sp, sem_a, sem_b):
    c = lax.axis_index("c")
    s = lax.axis_index("s")
    wid = s * NC + c
    pltpu.sync_copy(z_hbm.at[pl.ds(s * RPS, RPS)], acc.at[pl.ds(s * RPS, RPS)])
    pltpu.sync_copy(hs_hbm.at[pl.ds(s * RPS, RPS)],
                    hs_sp.at[pl.ds(s * RPS, RPS)])
    pltpu.sync_copy(row_hbm.at[wid], idxg_v)
    pltpu.sync_copy(col_hbm.at[wid], idxs_v)
    pltpu.sync_copy(w_hbm.at[wid], w_v)
    plsc.subcore_barrier()

    def gather_start(ch, buf, sem):
        pltpu.async_copy(hs_sp.at[idxg_v.at[ch]], buf, sem)

    def gather_wait(ch, buf, sem):
        pltpu.make_async_copy(hs_sp.at[idxg_v.at[ch]], buf, sem).wait()

    def process(ch, buf):
        for g in range(CH // 16):
            wvec = w_v[ch, pl.ds(g * 16, 16)]
            for k in range(16):
                i = g * 16 + k
                buf[i, :] = buf[i, :] * wvec[k]
        pltpu.sync_copy(buf, acc.at[idxs_v.at[ch]], add=True)

    gather_start(0, rows_a, sem_a)

    def pair(t, _):
        ch0 = 2 * t
        gather_start(ch0 + 1, rows_b, sem_b)
        gather_wait(ch0, rows_a, sem_a)
        process(ch0, rows_a)
        gather_start(ch0 + 2, rows_a, sem_a)
        gather_wait(ch0 + 1, rows_b, sem_b)
        process(ch0 + 1, rows_b)
        return _

    lax.fori_loop(0, (NCH - 1) // 2, pair, 0)
    gather_wait(NCH - 1, rows_a, sem_a)
    process(NCH - 1, rows_a)
    plsc.subcore_barrier()
    pltpu.sync_copy(acc.at[pl.ds(s * RPS, RPS)],
                    out_hbm.at[c, pl.ds(s * RPS, RPS)])


def _pre_body(x_ref, w1_ref, p0_ref, p1_ref, m_ref, dinv_ref, hs_ref):
    deg = p0_ref[...] + p1_ref[...] + m_ref[...]
    dinv = jnp.where(deg > 0, lax.rsqrt(deg), 0.0)
    dinv_ref[...] = dinv
    h = jnp.dot(x_ref[...], w1_ref[...], preferred_element_type=jnp.float32)
    hs_ref[...] = dinv * h


def _tc_pre(x_pad, w1, p0, p1, m):
    return pl.pallas_call(
        _pre_body,
        out_shape=[jax.ShapeDtypeStruct((NPAD, 1), jnp.float32),
                   jax.ShapeDtypeStruct((NPAD, D), jnp.float32)],
    )(x_pad, w1, p0, p1, m)


def _mid_body(a0_ref, a1_ref, hs_ref, dinv_ref, b_ref, wn_ref, out_ref):
    dinv = dinv_ref[...]
    y = jnp.maximum(
        dinv * (a0_ref[...] + a1_ref[...] + hs_ref[...]) + b_ref[...], 0.0)
    out_ref[...] = dinv * jnp.dot(y, wn_ref[...],
                                  preferred_element_type=jnp.float32)


def _tc_mid(a0, a1, hs, dinv, b, wn):
    return pl.pallas_call(
        _mid_body,
        out_shape=jax.ShapeDtypeStruct((NPAD, D), jnp.float32),
    )(a0, a1, hs, dinv, b, wn)


def _fin_body(a0_ref, a1_ref, hs_ref, dinv_ref, b_ref, out_ref):
    out_ref[...] = jnp.maximum(
        dinv_ref[...] * (a0_ref[...] + a1_ref[...] + hs_ref[...])
        + b_ref[...], 0.0)


def _tc_fin(a0, a1, hs, dinv, b):
    return pl.pallas_call(
        _fin_body,
        out_shape=jax.ShapeDtypeStruct((NPAD, D), jnp.float32),
    )(a0, a1, hs, dinv, b)


def _pad16(w):
    out = jnp.zeros((D, D), jnp.float32)
    return out.at[:w.shape[0], :w.shape[1]].set(w)


def kernel(x, edge_index, edge_weight, W1, b1, W2, b2, W3, b3, W4, b4):
    pad = EPT_PAD - EPT
    row = jnp.pad(edge_index[0].reshape(TILES, EPT),
                  ((0, 0), (0, pad))).reshape(TILES, NCH, CH)
    col = jnp.pad(edge_index[1].reshape(TILES, EPT),
                  ((0, 0), (0, pad)),
                  constant_values=NPAD - 1).reshape(TILES, NCH, CH)
    ew = jnp.pad(edge_weight.reshape(TILES, EPT),
                 ((0, 0), (0, pad))).reshape(TILES, NCH, CH)
    x_pad = jnp.zeros((NPAD, 128), jnp.float32).at[:N].set(x)
    z_n = jnp.zeros((NPAD,), jnp.float32)
    z_nd = jnp.zeros((NPAD, D), jnp.float32)
    m = (jnp.arange(NPAD) < N).astype(jnp.float32)[:, None]

    degp = _sc_degree(col, ew, z_n)
    p0 = degp[0][:, None]
    p1 = degp[1][:, None]
    dinv, hs = _tc_pre(x_pad, W1, p0, p1, m)

    w2p, w3p, w4p = _pad16(W2), _pad16(W3), _pad16(W4)
    b1p = b1[None, :]
    b2p = jnp.zeros((1, D), jnp.float32).at[0, :b2.shape[0]].set(b2)
    b3p = jnp.zeros((1, D), jnp.float32).at[0, :b3.shape[0]].set(b3)
    b4p = jnp.zeros((1, D), jnp.float32).at[0, :b4.shape[0]].set(b4)

    for bi, wn in ((b1p, w2p), (b2p, w3p), (b3p, w4p)):
        acc = _sc_edge(hs, row, col, ew, z_nd)
        hs = _tc_mid(acc[0], acc[1], hs, dinv, bi, wn)

    acc = _sc_edge(hs, row, col, ew, z_nd)
    y = _tc_fin(acc[0], acc[1], hs, dinv, b4p)
    return y[:N, :8]


# speedup vs baseline: 46.1954x; 1.0463x over previous
import functools
import jax
import jax.numpy as jnp
from jax import lax
from jax.experimental import pallas as pl
from jax.experimental.pallas import tpu as pltpu
from jax.experimental.pallas import tpu_sc as plsc

N = 10000
NPAD = 10240
E = 320000
D = 16
NC = 2
NS = 16
TILES = NC * NS
EPT = E // TILES
CH = 128
NCH = 79
EPT_PAD = NCH * CH
RPS = NPAD // NS

_mesh = plsc.VectorSubcoreMesh(core_axis_name="c", subcore_axis_name="s")


@functools.partial(
    pl.kernel, mesh=_mesh,
    compiler_params=pltpu.CompilerParams(use_tc_tiling_on_sc=False),
    out_type=jax.ShapeDtypeStruct((NC, NPAD), jnp.float32),
    scratch_types=[
        pltpu.VMEM((NCH, CH), jnp.int32),
        pltpu.VMEM((NCH, CH), jnp.float32),
        pltpu.VMEM_SHARED((NPAD,), jnp.float32),
        pltpu.SemaphoreType.DMA,
    ],
)
def _sc_degree(col_hbm, w_hbm, z_hbm, out_hbm, idx_v, w_v, acc, sem):
    c = lax.axis_index("c")
    s = lax.axis_index("s")
    wid = s * NC + c
    sl = pl.ds(s * RPS, RPS)
    c1 = pltpu.async_copy(z_hbm.at[sl], acc.at[sl], sem)
    c2 = pltpu.async_copy(col_hbm.at[wid], idx_v, sem)
    c3 = pltpu.async_copy(w_hbm.at[wid], w_v, sem)
    c1.wait(); c2.wait(); c3.wait()
    plsc.subcore_barrier()

    def body(ch, _):
        pltpu.sync_copy(w_v.at[ch], acc.at[idx_v.at[ch]], add=True)
        return _

    lax.fori_loop(0, NCH, body, 0)
    plsc.subcore_barrier()
    pltpu.sync_copy(acc.at[pl.ds(s * RPS, RPS)],
                    out_hbm.at[c, pl.ds(s * RPS, RPS)])


@functools.partial(
    pl.kernel, mesh=_mesh,
    compiler_params=pltpu.CompilerParams(use_tc_tiling_on_sc=False),
    out_type=jax.ShapeDtypeStruct((NC, NPAD, D), jnp.float32),
    scratch_types=[
        pltpu.VMEM((NCH, CH), jnp.int32),
        pltpu.VMEM((NCH, CH), jnp.int32),
        pltpu.VMEM((NCH, CH), jnp.float32),
        pltpu.VMEM((CH, D), jnp.float32),
        pltpu.VMEM((CH, D), jnp.float32),
        pltpu.VMEM_SHARED((NPAD, D), jnp.float32),
        pltpu.VMEM_SHARED((NPAD, D), jnp.float32),
        pltpu.SemaphoreType.DMA,
        pltpu.SemaphoreType.DMA,
    ],
)
def _sc_edge(hs_hbm, row_hbm, col_hbm, w_hbm, z_hbm, out_hbm,
             idxg_v, idxs_v, w_v, rows_a, rows_b, acc, hs_sp, sem_a, sem_b):
    c = lax.axis_index("c")
    s = lax.axis_index("s")
    wid = s * NC + c
    sl = pl.ds(s * RPS, RPS)
    c1 = pltpu.async_copy(z_hbm.at[sl], acc.at[sl], sem_a)
    c2 = pltpu.async_copy(hs_hbm.at[sl], hs_sp.at[sl], sem_a)
    c3 = pltpu.async_copy(row_hbm.at[wid], idxg_v, sem_b)
    c4 = pltpu.async_copy(col_hbm.at[wid], idxs_v, sem_b)
    c5 = pltpu.async_copy(w_hbm.at[wid], w_v, sem_b)
    c1.wait(); c2.wait(); c3.wait(); c4.wait(); c5.wait()
    plsc.subcore_barrier()

    def gather_start(ch, buf, sem):
        pltpu.async_copy(hs_sp.at[idxg_v.at[ch]], buf, sem)

    def gather_wait(ch, buf, sem):
        pltpu.make_async_copy(hs_sp.at[idxg_v.at[ch]], buf, sem).wait()

    def process(ch, buf):
        for g in range(CH // 16):
            wvec = w_v[ch, pl.ds(g * 16, 16)]
            for k in range(16):
                i = g * 16 + k
                buf[i, :] = buf[i, :] * wvec[k]
        pltpu.sync_copy(buf, acc.at[idxs_v.at[ch]], add=True)

    gather_start(0, rows_a, sem_a)

    def pair(t, _):
        ch0 = 2 * t
        gather_start(ch0 + 1, rows_b, sem_b)
        gather_wait(ch0, rows_a, sem_a)
        process(ch0, rows_a)
        gather_start(ch0 + 2, rows_a, sem_a)
        gather_wait(ch0 + 1, rows_b, sem_b)
        process(ch0 + 1, rows_b)
        return _

    lax.fori_loop(0, (NCH - 1) // 2, pair, 0)
    gather_wait(NCH - 1, rows_a, sem_a)
    process(NCH - 1, rows_a)
    plsc.subcore_barrier()
    pltpu.sync_copy(acc.at[pl.ds(s * RPS, RPS)],
                    out_hbm.at[c, pl.ds(s * RPS, RPS)])


def _pre_body(x_ref, w1_ref, p0_ref, p1_ref, m_ref, dinv_ref, hs_ref):
    deg = p0_ref[...] + p1_ref[...] + m_ref[...]
    dinv = jnp.where(deg > 0, lax.rsqrt(deg), 0.0)
    dinv_ref[...] = dinv
    h = jnp.dot(x_ref[...], w1_ref[...], preferred_element_type=jnp.float32)
    hs_ref[...] = dinv * h


def _tc_pre(x_pad, w1, p0, p1, m):
    return pl.pallas_call(
        _pre_body,
        out_shape=[jax.ShapeDtypeStruct((NPAD, 1), jnp.float32),
                   jax.ShapeDtypeStruct((NPAD, D), jnp.float32)],
    )(x_pad, w1, p0, p1, m)


def _mid_body(a0_ref, a1_ref, hs_ref, dinv_ref, b_ref, wn_ref, out_ref):
    dinv = dinv_ref[...]
    y = jnp.maximum(
        dinv * (a0_ref[...] + a1_ref[...] + hs_ref[...]) + b_ref[...], 0.0)
    out_ref[...] = dinv * jnp.dot(y, wn_ref[...],
                                  preferred_element_type=jnp.float32)


def _tc_mid(a0, a1, hs, dinv, b, wn):
    return pl.pallas_call(
        _mid_body,
        out_shape=jax.ShapeDtypeStruct((NPAD, D), jnp.float32),
    )(a0, a1, hs, dinv, b, wn)


def _fin_body(a0_ref, a1_ref, hs_ref, dinv_ref, b_ref, out_ref):
    out_ref[...] = jnp.maximum(
        dinv_ref[...] * (a0_ref[...] + a1_ref[...] + hs_ref[...])
        + b_ref[...], 0.0)


def _tc_fin(a0, a1, hs, dinv, b):
    return pl.pallas_call(
        _fin_body,
        out_shape=jax.ShapeDtypeStruct((NPAD, D), jnp.float32),
    )(a0, a1, hs, dinv, b)


def _pad16(w):
    out = jnp.zeros((D, D), jnp.float32)
    return out.at[:w.shape[0], :w.shape[1]].set(w)


def kernel(x, edge_index, edge_weight, W1, b1, W2, b2, W3, b3, W4, b4):
    pad = EPT_PAD - EPT
    row = jnp.pad(edge_index[0].reshape(TILES, EPT),
                  ((0, 0), (0, pad))).reshape(TILES, NCH, CH)
    col = jnp.pad(edge_index[1].reshape(TILES, EPT),
                  ((0, 0), (0, pad)),
                  constant_values=NPAD - 1).reshape(TILES, NCH, CH)
    ew = jnp.pad(edge_weight.reshape(TILES, EPT),
                 ((0, 0), (0, pad))).reshape(TILES, NCH, CH)
    x_pad = jnp.zeros((NPAD, 128), jnp.float32).at[:N].set(x)
    z_n = jnp.zeros((NPAD,), jnp.float32)
    z_nd = jnp.zeros((NPAD, D), jnp.float32)
    m = (jnp.arange(NPAD) < N).astype(jnp.float32)[:, None]

    degp = _sc_degree(col, ew, z_n)
    p0 = degp[0][:, None]
    p1 = degp[1][:, None]
    dinv, hs = _tc_pre(x_pad, W1, p0, p1, m)

    w2p, w3p, w4p = _pad16(W2), _pad16(W3), _pad16(W4)
    b1p = b1[None, :]
    b2p = jnp.zeros((1, D), jnp.float32).at[0, :b2.shape[0]].set(b2)
    b3p = jnp.zeros((1, D), jnp.float32).at[0, :b3.shape[0]].set(b3)
    b4p = jnp.zeros((1, D), jnp.float32).at[0, :b4.shape[0]].set(b4)

    for bi, wn in ((b1p, w2p), (b2p, w3p), (b3p, w4p)):
        acc = _sc_edge(hs, row, col, ew, z_nd)
        hs = _tc_mid(acc[0], acc[1], hs, dinv, bi, wn)

    acc = _sc_edge(hs, row, col, ew, z_nd)
    y = _tc_fin(acc[0], acc[1], hs, dinv, b4p)
    return y[:N, :8]
